# baseline (speedup 1.0000x reference)
import functools

import jax
import jax.numpy as jnp
import numpy as np
from jax import lax
from jax.experimental import pallas as pl
from jax.experimental.pallas import tpu as pltpu
from jax.experimental.pallas import tpu_sc as plsc

_NUM_CLASSES = 80
_C = _NUM_CLASSES + 6
_G = 64
_GG = _G * _G
_NA = 18
_B = 8
_BA = _B * _NA
_SCALE_XY = 1.05
_STRIDE = 8.0
_PI6 = 0.5235987755982988

_N_SC = 40
_N_TC = _BA - _N_SC

_CHUNK = 256
_N_CH = _GG // _CHUNK
_N_WORKERS = 32
_TASKS_PER_W = (_N_SC * _N_CH) // _N_WORKERS


def _anchor_consts(ba):
    a = ba % _NA
    ai = a // 6
    aj = a % 6
    aw8 = jnp.where(ai == 0, 12.0, jnp.where(ai == 1, 19.0, 40.0))
    ah8 = jnp.where(ai == 0, 16.0, jnp.where(ai == 1, 36.0, 28.0))
    aa = (aj.astype(jnp.float32) - 2.0) * np.float32(_PI6)
    return aw8, ah8, aa


def _sig(v):
    return 1.0 / (1.0 + jnp.exp(-v))


def _tc_body(x_ref, o_ref):
    aw8, ah8, aa = _anchor_consts(pl.program_id(0))
    v = x_ref[0]
    ci = lax.broadcasted_iota(jnp.int32, (_C, _GG), 0)
    gi = lax.broadcasted_iota(jnp.int32, (_C, _GG), 1)
    gx = (gi % _G).astype(jnp.float32)
    gy = (gi // _G).astype(jnp.float32)
    s = _sig(v)
    ex = jnp.exp(v)
    sxy = s * np.float32(_SCALE_XY * _STRIDE)
    off = np.float32((_SCALE_XY - 1.0) / 2.0 * _STRIDE)
    r = jnp.where(
        ci == 0, sxy + gx * np.float32(_STRIDE) - off,
        jnp.where(
            ci == 1, sxy + gy * np.float32(_STRIDE) - off,
            jnp.where(
                ci == 2, ex * aw8,
                jnp.where(ci == 3, ex * ah8, jnp.where(ci == 4, v + aa, s)))))
    o_ref[0] = r.T


def _tc_decode(x):
    return pl.pallas_call(
        _tc_body,
        grid=(_N_TC,),
        in_specs=[pl.BlockSpec((1, _C, _GG), lambda i: (i, 0, 0))],
        out_specs=pl.BlockSpec((1, _GG, _C), lambda i: (i, 0, 0)),
        out_shape=jax.ShapeDtypeStruct((_N_TC, _GG, _C), jnp.float32),
    )(x)


_mesh = plsc.VectorSubcoreMesh(core_axis_name="c", subcore_axis_name="s")


@functools.partial(
    pl.kernel,
    mesh=_mesh,
    out_type=jax.ShapeDtypeStruct((_N_SC, _N_CH, _CHUNK * _C), jnp.float32),
    scratch_types=[
        pltpu.VMEM((_C, _CHUNK), jnp.float32),
        pltpu.VMEM((_C, _CHUNK), jnp.float32),
        pltpu.VMEM((_CHUNK * _C,), jnp.float32),
        pltpu.VMEM((_CHUNK * _C,), jnp.float32),
        pltpu.SemaphoreType.DMA,
        pltpu.SemaphoreType.DMA,
        pltpu.SemaphoreType.DMA,
        pltpu.SemaphoreType.DMA,
    ],
    compiler_params=pltpu.CompilerParams(needs_layout_passes=False),
)
def _sc_decode(x_hbm, y_hbm, in0, in1, out0, out1, si0, si1, so0, so1):
    wid = lax.axis_index("s") * 2 + lax.axis_index("c")
    lane = lax.iota(jnp.int32, 16)
    lanef = lane.astype(jnp.float32)
    in_bufs = (in0, in1)
    out_bufs = (out0, out1)
    in_sems = (si0, si1)
    out_sems = (so0, so1)

    def in_slice(k):
        t = wid + k * _N_WORKERS
        ba = _N_TC + t // _N_CH
        return x_hbm.at[ba, :, pl.ds((t % _N_CH) * _CHUNK, _CHUNK)]

    def out_slice(k):
        t = wid + k * _N_WORKERS
        return y_hbm.at[t // _N_CH, t % _N_CH]

    pltpu.async_copy(in_slice(0), in0, si0)
    pltpu.async_copy(in_slice(1), in1, si1)

    @pl.loop(0, _TASKS_PER_W, step=2)
    def _task_pair(kk):
        for b in range(2):
            k = kk + b
            t = wid + k * _N_WORKERS
            ba = _N_TC + t // _N_CH
            g0 = (t % _N_CH) * _CHUNK
            aw8, ah8, aa = _anchor_consts(ba)
            in_v = in_bufs[b]
            out_v = out_bufs[b]

            pltpu.make_async_copy(in_slice(k), in_v, in_sems[b]).wait()

            @pl.when(kk >= 2)
            def _():
                pltpu.make_async_copy(out_v, out_slice(k), out_sems[b]).wait()

            @plsc.parallel_loop(0, _CHUNK // 16, unroll=2)
            def _jloop(j):
                gbase = g0 + j * 16
                gxf = (gbase % _G).astype(jnp.float32) + lanef
                gyf = (gbase // _G).astype(jnp.float32)
                gl86 = (j * 16 + lane) * _C
                for c in range(5):
                    v = in_v[c, pl.ds(j * 16, 16)]
                    if c == 0:
                        r = _sig(v) * 8.4 + (gxf * 8.0 - 0.2)
                    elif c == 1:
                        r = _sig(v) * 8.4 + (gyf * 8.0 - 0.2)
                    elif c == 2:
                        r = jnp.exp(v) * aw8
                    elif c == 3:
                        r = jnp.exp(v) * ah8
                    else:
                        r = v + aa
                    plsc.store_scatter(out_v, [gl86 + c], r)

            @plsc.parallel_loop(5 * (_CHUNK // 16), _C * (_CHUNK // 16),
                                unroll=8)
            def _mloop(m):
                p0 = m * 16
                c = p0 // _CHUNK
                gl = p0 % _CHUNK
                v = in_v[c, pl.ds(gl, 16)]
                r = _sig(v)
                plsc.store_scatter(out_v, [(gl + lane) * _C + c], r)

            pltpu.async_copy(out_v, out_slice(k), out_sems[b])

            @pl.when(k + 2 < _TASKS_PER_W)
            def _():
                pltpu.async_copy(in_slice(k + 2), in_v, in_sems[b])

    pltpu.make_async_copy(out0, out_slice(_TASKS_PER_W - 2), so0).wait()
    pltpu.make_async_copy(out1, out_slice(_TASKS_PER_W - 1), so1).wait()


def kernel(output):
    x = output.reshape(_BA, _C, _GG)
    o_tc = _tc_decode(x)
    o_sc = _sc_decode(x)
    flat = jnp.concatenate(
        [o_tc.reshape(_N_TC, _GG * _C),
         o_sc.reshape(_N_SC, _GG * _C)], axis=0)
    return flat.reshape(_B, _NA * _GG, _C)


# speedup vs baseline: 2.4293x; 2.4293x over previous
import jax
import jax.numpy as jnp
import numpy as np
from jax import lax
from jax.experimental import pallas as pl

_NUM_CLASSES = 80
_C = _NUM_CLASSES + 6
_G = 64
_GG = _G * _G
_NA = 18
_SCALE_XY = 1.05
_STRIDE = 8.0
_PI6 = 0.5235987755982988


def _body(x_ref, o_ref):
    ba = pl.program_id(0)
    a = ba % _NA
    ai = a // 6
    aj = a % 6
    aw8 = jnp.where(ai == 0, 12.0, jnp.where(ai == 1, 19.0, 40.0))
    ah8 = jnp.where(ai == 0, 16.0, jnp.where(ai == 1, 36.0, 28.0))
    aa = (aj.astype(jnp.float32) - 2.0) * np.float32(_PI6)

    v = x_ref[0]
    ci = lax.broadcasted_iota(jnp.int32, (_C, _G, _G), 0)
    gy = lax.broadcasted_iota(jnp.int32, (_C, _G, _G), 1).astype(jnp.float32)
    gx = lax.broadcasted_iota(jnp.int32, (_C, _G, _G), 2).astype(jnp.float32)
    s = 1.0 / (1.0 + jnp.exp(-v))
    ex = jnp.exp(v)
    sxy = s * np.float32(_SCALE_XY * _STRIDE)
    off = np.float32((_SCALE_XY - 1.0) / 2.0 * _STRIDE)
    r = jnp.where(
        ci == 0, sxy + gx * np.float32(_STRIDE) - off,
        jnp.where(
            ci == 1, sxy + gy * np.float32(_STRIDE) - off,
            jnp.where(
                ci == 2, ex * aw8,
                jnp.where(ci == 3, ex * ah8, jnp.where(ci == 4, v + aa, s)))))
    o_ref[0] = jnp.transpose(r, (1, 2, 0))


def kernel(output):
    B = output.shape[0]
    ba_total = B * _NA
    out = pl.pallas_call(
        _body,
        grid=(ba_total,),
        in_specs=[
            pl.BlockSpec((1, _C, _G, _G),
                         lambda i: (i // _NA, i % _NA, 0, 0)),
        ],
        out_specs=pl.BlockSpec((1, _G, _G, _C), lambda i: (i, 0, 0, 0)),
        out_shape=jax.ShapeDtypeStruct((ba_total, _G, _G, _C), jnp.float32),
    )(output)
    return out.reshape(B, _NA * _GG, _C)
